# baseline (speedup 1.0000x reference)
import functools
import jax
import jax.numpy as jnp
from jax import lax
from jax.experimental import pallas as pl
from jax.experimental.pallas import tpu as pltpu

K = 32
TM = 64
TM2 = 128


def _enc_body(x_ref, we_ref, lat_ref):
    x = x_ref[...]
    we = we_ref[...]
    latent = lax.dot_general(
        x, we, (((1,), (1,)), ((), ())),
        preferred_element_type=jnp.float32,
    )
    latent = jnp.maximum(latent, 0.0)
    H = latent.shape[1]

    neg = jnp.float32(-jnp.inf)

    cmax = jnp.max(latent.reshape(TM, H // 128, 128), axis=2)

    def cstep(_, carry):
        work, _ = carry
        m = jnp.max(work, axis=1, keepdims=True)
        work = jnp.where(work >= m, neg, work)
        return work, m

    _, lower = lax.fori_loop(
        0, K, cstep, (cmax, jnp.zeros((TM, 1), jnp.float32))
    )
    rowmax = jnp.max(cmax, axis=1, keepdims=True)

    lo0 = jnp.maximum(lax.bitcast_convert_type(lower, jnp.int32), 0)
    hi0 = lax.bitcast_convert_type(rowmax, jnp.int32) + 1
    kf = jnp.float32(K)

    def bstep(_, carry):
        lo, hi = carry
        mid = lo + ((hi - lo) >> 1)
        midf = lax.bitcast_convert_type(mid, jnp.float32)
        cnt = jnp.sum(
            jnp.where(latent >= midf, 1.0, 0.0), axis=1, keepdims=True
        )
        p = cnt >= kf
        lo = jnp.where(p, mid, lo)
        hi = jnp.where(p, hi, mid)
        return lo, hi

    lo_f, _ = lax.fori_loop(0, 31, bstep, (lo0, hi0))
    thresh = lax.bitcast_convert_type(lo_f, jnp.float32)

    lat_ref[...] = jnp.where(latent >= thresh, latent, 0.0)


def _dec_body(lat_ref, wd_ref, rec_ref):
    wd = wd_ref[...]
    rec_ref[...] = lax.dot_general(
        lat_ref[...], wd, (((1,), (1,)), ((), ())),
        preferred_element_type=jnp.float32,
    )


def kernel(x, W_enc, W_dec):
    N, D = x.shape
    H = W_enc.shape[0]

    latent_sparse = pl.pallas_call(
        _enc_body,
        grid=(N // TM,),
        in_specs=[
            pl.BlockSpec((TM, D), lambda i: (i, 0)),
            pl.BlockSpec((H, D), lambda i: (0, 0)),
        ],
        out_specs=pl.BlockSpec((TM, H), lambda i: (i, 0)),
        out_shape=jax.ShapeDtypeStruct((N, H), jnp.float32),
        compiler_params=pltpu.CompilerParams(
            dimension_semantics=("arbitrary",),
        ),
    )(x, W_enc)

    recon = pl.pallas_call(
        _dec_body,
        grid=(N // TM2,),
        in_specs=[
            pl.BlockSpec((TM2, H), lambda i: (i, 0)),
            pl.BlockSpec((D, H), lambda i: (0, 0)),
        ],
        out_specs=pl.BlockSpec((TM2, D), lambda i: (i, 0)),
        out_shape=jax.ShapeDtypeStruct((N, D), jnp.float32),
        compiler_params=pltpu.CompilerParams(
            dimension_semantics=("arbitrary",),
        ),
    )(latent_sparse, W_dec)

    return (latent_sparse, recon)


# speedup vs baseline: 3.2516x; 3.2516x over previous
import functools
import jax
import jax.numpy as jnp
from jax import lax
from jax.experimental import pallas as pl
from jax.experimental.pallas import tpu as pltpu

K = 32
TM = 128
TM2 = 128


def _enc_body(x_ref, we_ref, lat_ref):
    x = x_ref[...]
    we = we_ref[...]
    latent_t = lax.dot_general(
        we, x, (((1,), (1,)), ((), ())),
        preferred_element_type=jnp.float32,
    )
    latent_t = jnp.maximum(latent_t, 0.0)
    H = latent_t.shape[0]
    lat3 = latent_t.reshape(H // 128, 128, TM)

    neg = jnp.float32(-jnp.inf)

    def rep(col):
        return lax.transpose(jnp.broadcast_to(col, (TM, 128)), (1, 0))

    cmax = lax.transpose(jnp.max(lat3, axis=1), (1, 0))

    def cstep(_, carry):
        work, _ = carry
        m = jnp.max(work, axis=1, keepdims=True)
        work = jnp.where(work >= m, neg, work)
        return work, m

    _, lower = lax.fori_loop(
        0, K, cstep, (cmax, jnp.zeros((TM, 1), jnp.float32))
    )
    rowmax = jnp.max(cmax, axis=1, keepdims=True)

    lo0 = jnp.maximum(lax.bitcast_convert_type(lower, jnp.int32), 0)
    hi0 = lax.bitcast_convert_type(rowmax, jnp.int32) + 1
    kf = jnp.float32(K)

    def bstep(_, carry):
        lo, hi = carry
        mid = lo + ((hi - lo) >> 1)
        midf = lax.bitcast_convert_type(mid, jnp.float32)
        ind = jnp.where(lat3 >= rep(midf)[None], 1.0, 0.0)
        c1 = jnp.sum(ind, axis=0)
        cnt = jnp.sum(lax.transpose(c1, (1, 0)), axis=1,
                      keepdims=True)
        p = cnt >= kf
        lo = jnp.where(p, mid, lo)
        hi = jnp.where(p, hi, mid)
        return lo, hi

    lo_f, _ = lax.fori_loop(0, 31, bstep, (lo0, hi0))
    thresh = lax.bitcast_convert_type(lo_f, jnp.float32)

    sparse_t = jnp.where(lat3 >= rep(thresh)[None], lat3, 0.0)
    sparse_t = sparse_t.reshape(H, TM)
    lat_ref[...] = lax.transpose(sparse_t, (1, 0))


def _dec_body(lat_ref, wd_ref, rec_ref):
    wd = wd_ref[...]
    rec_ref[...] = lax.dot_general(
        lat_ref[...], wd, (((1,), (1,)), ((), ())),
        preferred_element_type=jnp.float32,
    )


def kernel(x, W_enc, W_dec):
    N, D = x.shape
    H = W_enc.shape[0]

    latent_sparse = pl.pallas_call(
        _enc_body,
        grid=(N // TM,),
        in_specs=[
            pl.BlockSpec((TM, D), lambda i: (i, 0)),
            pl.BlockSpec((H, D), lambda i: (0, 0)),
        ],
        out_specs=pl.BlockSpec((TM, H), lambda i: (i, 0)),
        out_shape=jax.ShapeDtypeStruct((N, H), jnp.float32),
        compiler_params=pltpu.CompilerParams(
            dimension_semantics=("arbitrary",),
        ),
    )(x, W_enc)

    recon = pl.pallas_call(
        _dec_body,
        grid=(N // TM2,),
        in_specs=[
            pl.BlockSpec((TM2, H), lambda i: (i, 0)),
            pl.BlockSpec((D, H), lambda i: (0, 0)),
        ],
        out_specs=pl.BlockSpec((TM2, D), lambda i: (i, 0)),
        out_shape=jax.ShapeDtypeStruct((N, D), jnp.float32),
        compiler_params=pltpu.CompilerParams(
            dimension_semantics=("arbitrary",),
        ),
    )(latent_sparse, W_dec)

    return (latent_sparse, recon)


# speedup vs baseline: 4.4476x; 1.3678x over previous
import functools
import jax
import jax.numpy as jnp
from jax import lax
from jax.experimental import pallas as pl
from jax.experimental.pallas import tpu as pltpu

K = 32
TM = 128
TM2 = 128


def _enc_body(x_ref, we_ref, lat_ref):
    x = x_ref[...]
    we = we_ref[...]
    latent_t = lax.dot_general(
        we, x, (((1,), (1,)), ((), ())),
        preferred_element_type=jnp.float32,
    )
    latent_t = jnp.maximum(latent_t, 0.0)
    H = latent_t.shape[0]
    lat3 = latent_t.reshape(H // 128, 128, TM)

    neg = jnp.float32(-jnp.inf)

    def rep(col):
        return lax.transpose(jnp.broadcast_to(col, (TM, 128)), (1, 0))

    cmax = lax.transpose(jnp.max(lat3, axis=1), (1, 0))

    def cstep(_, carry):
        work, _ = carry
        m = jnp.max(work, axis=1, keepdims=True)
        work = jnp.where(work >= m, neg, work)
        return work, m

    _, lower = lax.fori_loop(
        0, K, cstep, (cmax, jnp.zeros((TM, 1), jnp.float32))
    )
    rowmax = jnp.max(cmax, axis=1, keepdims=True)

    lo0 = jnp.maximum(lax.bitcast_convert_type(lower, jnp.int32), 0)
    hi0 = lax.bitcast_convert_type(rowmax, jnp.int32) + 1
    kf = jnp.float32(K)

    def bcond(state):
        lo, hi, done = state
        return jnp.sum(done) < TM

    def bstep(state):
        lo, hi, done = state
        mid = lo + ((hi - lo) >> 1)
        midf = lax.bitcast_convert_type(mid, jnp.float32)
        ind = jnp.where(lat3 >= rep(midf)[None], 1.0, 0.0)
        c1 = jnp.sum(ind, axis=0)
        cnt = jnp.sum(lax.transpose(c1, (1, 0)), axis=1,
                      keepdims=True)
        p = cnt >= kf
        live = done == 0
        nlo = jnp.where(jnp.logical_and(live, p), mid, lo)
        nhi = jnp.where(jnp.logical_and(live, jnp.logical_not(p)), mid, hi)
        hit = jnp.logical_and(live, cnt == kf)
        nlo = jnp.where(hit, mid, nlo)
        ndone = jnp.where(
            jnp.logical_or(hit, nhi - nlo <= 1), jnp.int32(1), done
        )
        return nlo, nhi, ndone

    dinit = jnp.where(hi0 - lo0 <= 1, jnp.int32(1), jnp.int32(0))
    lo_f, _, _ = lax.while_loop(bcond, bstep, (lo0, hi0, dinit))
    thresh = lax.bitcast_convert_type(lo_f, jnp.float32)

    sparse_t = jnp.where(lat3 >= rep(thresh)[None], lat3, 0.0)
    sparse_t = sparse_t.reshape(H, TM)
    lat_ref[...] = lax.transpose(sparse_t, (1, 0))


def _dec_body(lat_ref, wd_ref, rec_ref):
    wd = wd_ref[...]
    rec_ref[...] = lax.dot_general(
        lat_ref[...], wd, (((1,), (1,)), ((), ())),
        preferred_element_type=jnp.float32,
    )


def kernel(x, W_enc, W_dec):
    N, D = x.shape
    H = W_enc.shape[0]

    latent_sparse = pl.pallas_call(
        _enc_body,
        grid=(N // TM,),
        in_specs=[
            pl.BlockSpec((TM, D), lambda i: (i, 0)),
            pl.BlockSpec((H, D), lambda i: (0, 0)),
        ],
        out_specs=pl.BlockSpec((TM, H), lambda i: (i, 0)),
        out_shape=jax.ShapeDtypeStruct((N, H), jnp.float32),
        compiler_params=pltpu.CompilerParams(
            dimension_semantics=("arbitrary",),
        ),
    )(x, W_enc)

    recon = pl.pallas_call(
        _dec_body,
        grid=(N // TM2,),
        in_specs=[
            pl.BlockSpec((TM2, H), lambda i: (i, 0)),
            pl.BlockSpec((D, H), lambda i: (0, 0)),
        ],
        out_specs=pl.BlockSpec((TM2, D), lambda i: (i, 0)),
        out_shape=jax.ShapeDtypeStruct((N, D), jnp.float32),
        compiler_params=pltpu.CompilerParams(
            dimension_semantics=("arbitrary",),
        ),
    )(latent_sparse, W_dec)

    return (latent_sparse, recon)


# speedup vs baseline: 4.5771x; 1.0291x over previous
import functools
import jax
import jax.numpy as jnp
from jax import lax
from jax.experimental import pallas as pl
from jax.experimental.pallas import tpu as pltpu

K = 32
TM = 128
TM2 = 128


def _enc_body(x_ref, we_ref, lat_ref):
    x = x_ref[...]
    we = we_ref[...]
    latent_t = lax.dot_general(
        we, x, (((1,), (1,)), ((), ())),
        preferred_element_type=jnp.float32,
    )
    latent_t = jnp.maximum(latent_t, 0.0)
    H = latent_t.shape[0]
    lat3 = latent_t.reshape(H // 128, 128, TM)

    neg = jnp.float32(-jnp.inf)

    def rep(col):
        return lax.transpose(jnp.broadcast_to(col, (TM, 128)), (1, 0))

    cmax = lax.transpose(jnp.max(lat3, axis=1), (1, 0))
    rowmax = jnp.max(cmax, axis=1, keepdims=True)
    hi0 = lax.bitcast_convert_type(rowmax, jnp.int32) + 1
    kf = jnp.float32(K)

    def ccond(state):
        lo, hi, done = state
        return jnp.sum(done) < TM

    def cbody(state):
        lo, hi, done = state
        mid = lo + ((hi - lo) >> 1)
        midf = lax.bitcast_convert_type(mid, jnp.float32)
        cntc = jnp.sum(jnp.where(cmax >= midf, 1.0, 0.0), axis=1,
                       keepdims=True)
        p = cntc >= kf
        live = done == 0
        nlo = jnp.where(jnp.logical_and(live, p), mid, lo)
        nhi = jnp.where(jnp.logical_and(live, jnp.logical_not(p)), mid, hi)
        hit = jnp.logical_and(live, cntc == kf)
        nlo = jnp.where(hit, mid, nlo)
        ndone = jnp.where(
            jnp.logical_or(hit, nhi - nlo <= 1), jnp.int32(1), done
        )
        return nlo, nhi, ndone

    zlo = jnp.zeros((TM, 1), jnp.int32)
    cdone0 = jnp.where(hi0 <= 1, jnp.int32(1), jnp.int32(0))
    lo0, _, _ = lax.while_loop(ccond, cbody, (zlo, hi0, cdone0))


    def bcond(state):
        lo, hi, done = state
        return jnp.sum(done) < TM

    def bstep(state):
        lo, hi, done = state
        mid = lo + ((hi - lo) >> 1)
        midf = lax.bitcast_convert_type(mid, jnp.float32)
        ind = jnp.where(lat3 >= rep(midf)[None], 1.0, 0.0)
        c1 = jnp.sum(ind, axis=0)
        cnt = jnp.sum(lax.transpose(c1, (1, 0)), axis=1,
                      keepdims=True)
        p = cnt >= kf
        live = done == 0
        nlo = jnp.where(jnp.logical_and(live, p), mid, lo)
        nhi = jnp.where(jnp.logical_and(live, jnp.logical_not(p)), mid, hi)
        hit = jnp.logical_and(live, cnt == kf)
        nlo = jnp.where(hit, mid, nlo)
        ndone = jnp.where(
            jnp.logical_or(hit, nhi - nlo <= 1), jnp.int32(1), done
        )
        return nlo, nhi, ndone

    dinit = jnp.where(hi0 - lo0 <= 1, jnp.int32(1), jnp.int32(0))
    lo_f, _, _ = lax.while_loop(bcond, bstep, (lo0, hi0, dinit))
    thresh = lax.bitcast_convert_type(lo_f, jnp.float32)

    sparse_t = jnp.where(lat3 >= rep(thresh)[None], lat3, 0.0)
    sparse_t = sparse_t.reshape(H, TM)
    lat_ref[...] = lax.transpose(sparse_t, (1, 0))


def _dec_body(lat_ref, wd_ref, rec_ref):
    wd = wd_ref[...]
    rec_ref[...] = lax.dot_general(
        lat_ref[...], wd, (((1,), (1,)), ((), ())),
        preferred_element_type=jnp.float32,
    )


def kernel(x, W_enc, W_dec):
    N, D = x.shape
    H = W_enc.shape[0]

    latent_sparse = pl.pallas_call(
        _enc_body,
        grid=(N // TM,),
        in_specs=[
            pl.BlockSpec((TM, D), lambda i: (i, 0)),
            pl.BlockSpec((H, D), lambda i: (0, 0)),
        ],
        out_specs=pl.BlockSpec((TM, H), lambda i: (i, 0)),
        out_shape=jax.ShapeDtypeStruct((N, H), jnp.float32),
        compiler_params=pltpu.CompilerParams(
            dimension_semantics=("arbitrary",),
        ),
    )(x, W_enc)

    recon = pl.pallas_call(
        _dec_body,
        grid=(N // TM2,),
        in_specs=[
            pl.BlockSpec((TM2, H), lambda i: (i, 0)),
            pl.BlockSpec((D, H), lambda i: (0, 0)),
        ],
        out_specs=pl.BlockSpec((TM2, D), lambda i: (i, 0)),
        out_shape=jax.ShapeDtypeStruct((N, D), jnp.float32),
        compiler_params=pltpu.CompilerParams(
            dimension_semantics=("arbitrary",),
        ),
    )(latent_sparse, W_dec)

    return (latent_sparse, recon)


# speedup vs baseline: 4.6165x; 1.0086x over previous
import functools
import jax
import jax.numpy as jnp
from jax import lax
from jax.experimental import pallas as pl
from jax.experimental.pallas import tpu as pltpu

K = 32
TM = 128
TM2 = 128


def _enc_body(x_ref, we_ref, lat_ref):
    x = x_ref[...]
    we = we_ref[...]
    latent_t = lax.dot_general(
        we, x, (((1,), (1,)), ((), ())),
        preferred_element_type=jnp.float32,
    )
    latent_t = jnp.maximum(latent_t, 0.0)
    H = latent_t.shape[0]
    lat3 = latent_t.reshape(H // 128, 128, TM)

    neg = jnp.float32(-jnp.inf)

    def rep(col):
        return lax.transpose(jnp.broadcast_to(col, (TM, 128)), (1, 0))

    cmax = lax.transpose(jnp.max(lat3, axis=1), (1, 0))
    rowmax = jnp.max(cmax, axis=1, keepdims=True)
    hi0 = lax.bitcast_convert_type(rowmax, jnp.int32) + 1
    kf = jnp.float32(K)

    def ccond(state):
        lo, hi, done = state
        return jnp.sum(done) < TM

    def cbody(state):
        lo, hi, done = state
        mid = lo + ((hi - lo) >> 1)
        midf = lax.bitcast_convert_type(mid, jnp.float32)
        cntc = jnp.sum(jnp.where(cmax >= midf, 1.0, 0.0), axis=1,
                       keepdims=True)
        p = cntc >= kf
        live = done == 0
        nlo = jnp.where(jnp.logical_and(live, p), mid, lo)
        nhi = jnp.where(jnp.logical_and(live, jnp.logical_not(p)), mid, hi)
        hit = jnp.logical_and(live, cntc == kf)
        nlo = jnp.where(hit, mid, nlo)
        ndone = jnp.where(
            jnp.logical_or(hit, nhi - nlo <= 1), jnp.int32(1), done
        )
        return nlo, nhi, ndone

    zlo = jnp.zeros((TM, 1), jnp.int32)
    cdone0 = jnp.where(hi0 <= 1, jnp.int32(1), jnp.int32(0))
    lo0, _, _ = lax.while_loop(ccond, cbody, (zlo, hi0, cdone0))


    def sroll(a, sh):
        return jnp.concatenate([a[-sh:, :], a[:-sh, :]], axis=0)

    def repi(col):
        f = lax.bitcast_convert_type(col, jnp.float32)
        return lax.bitcast_convert_type(rep(f), jnp.int32)

    lo0r = repi(lo0)
    hi0r = repi(hi0)

    def bcond(state):
        lo, hi, done = state
        return jnp.sum(done) < TM * 128

    def bstep(state):
        lo, hi, done = state
        mid = lo + ((hi - lo) >> 1)
        midf = lax.bitcast_convert_type(mid, jnp.float32)
        cnt = jnp.zeros((128, TM), jnp.float32)
        for s in range(0, 96, 8):
            slab = lat3[s:s + 8]
            cnt = cnt + jnp.sum(
                jnp.where(slab >= midf[None], 1.0, 0.0), axis=0
            )
        for sh in (64, 32, 16, 8, 4, 2, 1):
            cnt = cnt + sroll(cnt, sh)
        p = cnt >= kf
        live = done == 0
        nlo = jnp.where(jnp.logical_and(live, p), mid, lo)
        nhi = jnp.where(jnp.logical_and(live, jnp.logical_not(p)), mid, hi)
        hit = jnp.logical_and(live, cnt == kf)
        nlo = jnp.where(hit, mid, nlo)
        ndone = jnp.where(
            jnp.logical_or(hit, nhi - nlo <= 1), jnp.int32(1), done
        )
        return nlo, nhi, ndone

    dinit = jnp.where(hi0r - lo0r <= 1, jnp.int32(1), jnp.int32(0))
    lo_f, _, _ = lax.while_loop(bcond, bstep, (lo0r, hi0r, dinit))
    thresh = lax.bitcast_convert_type(lo_f, jnp.float32)

    CH = 1024
    for c in range(H // CH):
        blk = lat3[c * (CH // 128):(c + 1) * (CH // 128)]
        sp = jnp.where(blk >= thresh[None], blk, 0.0).reshape(CH, TM)
        lat_ref[:, c * CH:(c + 1) * CH] = lax.transpose(sp, (1, 0))


def _dec_body(lat_ref, wd_ref, rec_ref):
    wd = wd_ref[...]
    rec_ref[...] = lax.dot_general(
        lat_ref[...], wd, (((1,), (1,)), ((), ())),
        preferred_element_type=jnp.float32,
    )


def kernel(x, W_enc, W_dec):
    N, D = x.shape
    H = W_enc.shape[0]

    latent_sparse = pl.pallas_call(
        _enc_body,
        grid=(N // TM,),
        in_specs=[
            pl.BlockSpec((TM, D), lambda i: (i, 0)),
            pl.BlockSpec((H, D), lambda i: (0, 0)),
        ],
        out_specs=pl.BlockSpec((TM, H), lambda i: (i, 0)),
        out_shape=jax.ShapeDtypeStruct((N, H), jnp.float32),
        compiler_params=pltpu.CompilerParams(
            dimension_semantics=("arbitrary",),
        ),
    )(x, W_enc)

    recon = pl.pallas_call(
        _dec_body,
        grid=(N // TM2,),
        in_specs=[
            pl.BlockSpec((TM2, H), lambda i: (i, 0)),
            pl.BlockSpec((D, H), lambda i: (0, 0)),
        ],
        out_specs=pl.BlockSpec((TM2, D), lambda i: (i, 0)),
        out_shape=jax.ShapeDtypeStruct((N, D), jnp.float32),
        compiler_params=pltpu.CompilerParams(
            dimension_semantics=("arbitrary",),
        ),
    )(latent_sparse, W_dec)

    return (latent_sparse, recon)
